# Initial kernel scaffold; baseline (speedup 1.0000x reference)
import jax
import jax.numpy as jnp
from jax.experimental import pallas as pl


def kernel(feats, ancestries, W_phk, b_phk, W_phv, b_phv, ph_query, W_ank, b_ank, W_anv, b_anv, ancestry_query):
    raise NotImplementedError("write your pallas kernel here")


# speedup vs baseline: 3.6806x; 3.6806x over previous
import functools

import jax
import jax.numpy as jnp
from jax.experimental import pallas as pl
from jax.experimental.pallas import tpu as pltpu

B = 1024
N = 100
D = 128
G = 8
ROWS = G * N


def _body(f_ref, oh_ref, phq_ref, Wphk_ref, bphk_ref, Wphv_ref, bphv_ref,
          anq_ref, Wank_ref, bank_ref, Wanv_ref, banv_ref,
          h_ref, wp_ref, wa_ref):
    f = f_ref[...]
    phq = phq_ref[...]
    anq = anq_ref[...]
    dn = (((1,), (0,)), ((), ()))
    dnt = (((1,), (1,)), ((), ()))

    qph = jax.lax.dot_general(phq, Wphk_ref[...], dn,
                              preferred_element_type=jnp.float32)
    cph = jnp.sum(phq * bphk_ref[...])
    AQ = jax.lax.dot_general(anq, Wank_ref[...], dn,
                             preferred_element_type=jnp.float32)
    can4 = jnp.sum(anq * bank_ref[...], axis=1, keepdims=True)
    oh = oh_ref[...]
    qa = jax.lax.dot_general(oh, AQ, dn,
                             preferred_element_type=jnp.float32)
    can = jax.lax.dot_general(oh, can4, dn,
                              preferred_element_type=jnp.float32)

    lp = jnp.sum(f * qph, axis=1, keepdims=True) + cph
    La = jax.lax.dot_general(f, qa, dnt,
                             preferred_element_type=jnp.float32)
    r = jax.lax.broadcasted_iota(jnp.int32, (ROWS, G), 0)
    g = jax.lax.broadcasted_iota(jnp.int32, (ROWS, G), 1)
    segt = (r // N == g).astype(jnp.float32)
    la = (jnp.sum(La * segt, axis=1, keepdims=True)
          + jax.lax.dot_general(segt, can, dn,
                                preferred_element_type=jnp.float32))
    wp = jax.nn.sigmoid(lp)
    wa = jax.nn.sigmoid(la)
    wp_ref[...] = wp
    wa_ref[...] = wa

    r2 = jax.lax.broadcasted_iota(jnp.int32, (G, ROWS), 1)
    g2 = jax.lax.broadcasted_iota(jnp.int32, (G, ROWS), 0)
    seg = (r2 // N == g2).astype(jnp.float32)
    sph = jax.lax.dot_general(seg, f * wp, dn,
                              preferred_element_type=jnp.float32)
    san = jax.lax.dot_general(seg, f * wa, dn,
                              preferred_element_type=jnp.float32)
    wsp = jax.lax.dot_general(seg, wp, dn,
                              preferred_element_type=jnp.float32)
    wsa = jax.lax.dot_general(seg, wa, dn,
                              preferred_element_type=jnp.float32)
    h_ref[...] = (jax.lax.dot_general(sph, Wphv_ref[...], dnt,
                                      preferred_element_type=jnp.float32)
                  + wsp * bphv_ref[...]
                  + jax.lax.dot_general(san, Wanv_ref[...], dnt,
                                        preferred_element_type=jnp.float32)
                  + wsa * banv_ref[...])


@functools.partial(jax.jit, static_argnames=())
def kernel(feats, ancestries, W_phk, b_phk, W_phv, b_phv, ph_query,
           W_ank, b_ank, W_anv, b_anv, ancestry_query):
    oh = (ancestries[:, None] == jnp.arange(4, dtype=jnp.int32)[None, :]
          ).astype(jnp.float32)
    full = lambda shape: pl.BlockSpec(shape, lambda i: (0, 0))
    grid = B // G
    h, wp, wa = pl.pallas_call(
        _body,
        grid=(grid,),
        in_specs=[
            pl.BlockSpec((ROWS, D), lambda i: (i, 0)),
            pl.BlockSpec((G, 4), lambda i: (i, 0)),
            full((1, D)),
            full((D, D)),
            full((1, D)),
            full((D, D)),
            full((1, D)),
            full((4, D)),
            full((D, D)),
            full((1, D)),
            full((D, D)),
            full((1, D)),
        ],
        out_specs=[
            pl.BlockSpec((G, D), lambda i: (i, 0)),
            pl.BlockSpec((ROWS, 1), lambda i: (i, 0)),
            pl.BlockSpec((ROWS, 1), lambda i: (i, 0)),
        ],
        out_shape=[
            jax.ShapeDtypeStruct((B, D), jnp.float32),
            jax.ShapeDtypeStruct((B * N, 1), jnp.float32),
            jax.ShapeDtypeStruct((B * N, 1), jnp.float32),
        ],
        compiler_params=pltpu.CompilerParams(
            dimension_semantics=("parallel",)),
    )(feats, oh, ph_query, W_phk, b_phk.reshape(1, D), W_phv,
      b_phv.reshape(1, D), ancestry_query, W_ank, b_ank.reshape(1, D),
      W_anv, b_anv.reshape(1, D))
    return (h, wp, wa)


# speedup vs baseline: 13.2190x; 3.5915x over previous
import functools

import jax
import jax.numpy as jnp
from jax.experimental import pallas as pl
from jax.experimental.pallas import tpu as pltpu

B = 1024
N = 100
D = 128
G = 64
GI = 8
NCH = G // GI
CROWS = GI * N
ROWS = G * N
GRID = B // G


def _body(f_ref, oh_ref, phq_ref, Wphk_ref, bphk_ref, Wphv_ref, bphv_ref,
          anq_ref, Wank_ref, bank_ref, Wanv_ref, banv_ref,
          h_ref, wp_ref, wa_ref):
    f = f_ref[...]
    phq = phq_ref[...]
    anq = anq_ref[...]
    dn = (((1,), (0,)), ((), ()))
    dnt = (((1,), (1,)), ((), ()))

    qph = jax.lax.dot_general(phq, Wphk_ref[...], dn,
                              preferred_element_type=jnp.float32)
    cph = jnp.sum(phq * bphk_ref[...])
    AQ = jax.lax.dot_general(anq, Wank_ref[...], dn,
                             preferred_element_type=jnp.float32)
    can4 = jnp.sum(anq * bank_ref[...], axis=1, keepdims=True)
    oh = oh_ref[...]
    qa = jax.lax.dot_general(oh, AQ, dn,
                             preferred_element_type=jnp.float32)
    can = jax.lax.dot_general(oh, can4, dn,
                              preferred_element_type=jnp.float32)

    rlane = jax.lax.broadcasted_iota(jnp.int32, (GI, CROWS), 1)
    gsub = jax.lax.broadcasted_iota(jnp.int32, (GI, CROWS), 0)
    seg8 = (rlane // N == gsub).astype(jnp.float32)

    pad = jnp.zeros((16 - GI - 1, D), jnp.float32)
    sph_l, san_l, wsp_l, wsa_l = [], [], [], []
    for c in range(NCH):
        fc = f[c * CROWS:(c + 1) * CROWS]
        qac = qa[c * GI:(c + 1) * GI]
        canc = can[c * GI:(c + 1) * GI]
        q16 = jnp.concatenate([qac, qph, pad], axis=0)
        L = jax.lax.dot_general(fc, q16, dnt,
                                preferred_element_type=jnp.float32)
        Lt = L.T
        la_t = jnp.sum((Lt[0:GI] + canc) * seg8, axis=0,
                       keepdims=True)
        lp_t = Lt[GI:GI + 1] + cph
        wp_t = jax.nn.sigmoid(lp_t)
        wa_t = jax.nn.sigmoid(la_t)
        wp_ref[0, 0, c * CROWS:(c + 1) * CROWS] = wp_t.reshape(CROWS)
        wa_ref[0, 0, c * CROWS:(c + 1) * CROWS] = wa_t.reshape(CROWS)
        Wp = seg8 * wp_t
        Wa = seg8 * wa_t
        sph_l.append(jax.lax.dot_general(Wp, fc, dn,
                                         preferred_element_type=jnp.float32))
        san_l.append(jax.lax.dot_general(Wa, fc, dn,
                                         preferred_element_type=jnp.float32))
        wsp_l.append(jnp.sum(Wp, axis=1, keepdims=True))
        wsa_l.append(jnp.sum(Wa, axis=1, keepdims=True))

    sph = jnp.concatenate(sph_l, axis=0)
    san = jnp.concatenate(san_l, axis=0)
    wsp = jnp.concatenate(wsp_l, axis=0)
    wsa = jnp.concatenate(wsa_l, axis=0)
    h_ref[...] = (jax.lax.dot_general(sph, Wphv_ref[...], dnt,
                                      preferred_element_type=jnp.float32)
                  + wsp * bphv_ref[...]
                  + jax.lax.dot_general(san, Wanv_ref[...], dnt,
                                        preferred_element_type=jnp.float32)
                  + wsa * banv_ref[...])


@functools.partial(jax.jit, static_argnames=())
def kernel(feats, ancestries, W_phk, b_phk, W_phv, b_phv, ph_query,
           W_ank, b_ank, W_anv, b_anv, ancestry_query):
    oh = (ancestries[:, None] == jnp.arange(4, dtype=jnp.int32)[None, :]
          ).astype(jnp.float32)
    full = lambda shape: pl.BlockSpec(shape, lambda i: (0, 0))
    h, wp, wa = pl.pallas_call(
        _body,
        grid=(GRID,),
        in_specs=[
            pl.BlockSpec((ROWS, D), lambda i: (i, 0)),
            pl.BlockSpec((G, 4), lambda i: (i, 0)),
            full((1, D)),
            full((D, D)),
            full((1, D)),
            full((D, D)),
            full((1, D)),
            full((4, D)),
            full((D, D)),
            full((1, D)),
            full((D, D)),
            full((1, D)),
        ],
        out_specs=[
            pl.BlockSpec((G, D), lambda i: (i, 0)),
            pl.BlockSpec((1, 1, ROWS), lambda i: (i, 0, 0)),
            pl.BlockSpec((1, 1, ROWS), lambda i: (i, 0, 0)),
        ],
        out_shape=[
            jax.ShapeDtypeStruct((B, D), jnp.float32),
            jax.ShapeDtypeStruct((GRID, 1, ROWS), jnp.float32),
            jax.ShapeDtypeStruct((GRID, 1, ROWS), jnp.float32),
        ],
        compiler_params=pltpu.CompilerParams(
            dimension_semantics=("parallel",)),
    )(feats, oh, ph_query, W_phk, b_phk.reshape(1, D), W_phv,
      b_phv.reshape(1, D), ancestry_query, W_ank, b_ank.reshape(1, D),
      W_anv, b_anv.reshape(1, D))
    return (h, wp.reshape(B * N, 1), wa.reshape(B * N, 1))


# speedup vs baseline: 21.5174x; 1.6278x over previous
import functools

import jax
import jax.numpy as jnp
from jax.experimental import pallas as pl
from jax.experimental.pallas import tpu as pltpu

B = 1024
N = 100
D = 128
G = 256
GI = 64
NCH = G // GI
CROWS = GI * N
ROWS = G * N
GRID = B // G


def _body(f_ref, oh_ref, phq_ref, Wphk_ref, bphk_ref, Wphv_ref, bphv_ref,
          anq_ref, Wank_ref, bank_ref, Wanv_ref, banv_ref,
          h_ref, wp_ref, wa_ref):
    f = f_ref[...]
    phq = phq_ref[...]
    anq = anq_ref[...]
    dn = (((1,), (0,)), ((), ()))
    dnt = (((1,), (1,)), ((), ()))

    qph = jax.lax.dot_general(phq, Wphk_ref[...], dn,
                              preferred_element_type=jnp.float32)
    cph = jnp.sum(phq * bphk_ref[...])
    AQ = jax.lax.dot_general(anq, Wank_ref[...], dn,
                             preferred_element_type=jnp.float32)
    can4 = jnp.sum(anq * bank_ref[...], axis=1, keepdims=True)
    oh = oh_ref[...]
    qa = jax.lax.dot_general(oh, AQ, dn,
                             preferred_element_type=jnp.float32)
    can = jax.lax.dot_general(oh, can4, dn,
                              preferred_element_type=jnp.float32)

    rlane = jax.lax.broadcasted_iota(jnp.int32, (GI, CROWS), 1)
    gsub = jax.lax.broadcasted_iota(jnp.int32, (GI, CROWS), 0)
    seg8 = (rlane // N == gsub).astype(jnp.float32)

    pad = jnp.zeros(((-(GI + 1)) % 8, D), jnp.float32)
    sph_l, san_l, wsp_l, wsa_l = [], [], [], []
    for c in range(NCH):
        fc = f[c * CROWS:(c + 1) * CROWS]
        qac = qa[c * GI:(c + 1) * GI]
        canc = can[c * GI:(c + 1) * GI]
        q16 = jnp.concatenate([qac, qph, pad], axis=0)
        L = jax.lax.dot_general(fc, q16, dnt,
                                preferred_element_type=jnp.float32)
        Lt = L.T
        la_t = jnp.sum((Lt[0:GI] + canc) * seg8, axis=0,
                       keepdims=True)
        lp_t = Lt[GI:GI + 1] + cph
        wp_t = jax.nn.sigmoid(lp_t)
        wa_t = jax.nn.sigmoid(la_t)
        wp_ref[0, c, :] = wp_t.reshape(CROWS)
        wa_ref[0, c, :] = wa_t.reshape(CROWS)
        Wp = seg8 * wp_t
        Wa = seg8 * wa_t
        sph_l.append(jax.lax.dot_general(Wp, fc, dn,
                                         preferred_element_type=jnp.float32))
        san_l.append(jax.lax.dot_general(Wa, fc, dn,
                                         preferred_element_type=jnp.float32))
        wsp_l.append(jnp.sum(Wp, axis=1, keepdims=True))
        wsa_l.append(jnp.sum(Wa, axis=1, keepdims=True))

    sph = jnp.concatenate(sph_l, axis=0)
    san = jnp.concatenate(san_l, axis=0)
    wsp = jnp.concatenate(wsp_l, axis=0)
    wsa = jnp.concatenate(wsa_l, axis=0)
    h_ref[...] = (jax.lax.dot_general(sph, Wphv_ref[...], dnt,
                                      preferred_element_type=jnp.float32)
                  + wsp * bphv_ref[...]
                  + jax.lax.dot_general(san, Wanv_ref[...], dnt,
                                        preferred_element_type=jnp.float32)
                  + wsa * banv_ref[...])


@functools.partial(jax.jit, static_argnames=())
def kernel(feats, ancestries, W_phk, b_phk, W_phv, b_phv, ph_query,
           W_ank, b_ank, W_anv, b_anv, ancestry_query):
    oh = (ancestries[:, None] == jnp.arange(4, dtype=jnp.int32)[None, :]
          ).astype(jnp.float32)
    full = lambda shape: pl.BlockSpec(shape, lambda i: (0, 0))
    h, wp, wa = pl.pallas_call(
        _body,
        grid=(GRID,),
        in_specs=[
            pl.BlockSpec((ROWS, D), lambda i: (i, 0)),
            pl.BlockSpec((G, 4), lambda i: (i, 0)),
            full((1, D)),
            full((D, D)),
            full((1, D)),
            full((D, D)),
            full((1, D)),
            full((4, D)),
            full((D, D)),
            full((1, D)),
            full((D, D)),
            full((1, D)),
        ],
        out_specs=[
            pl.BlockSpec((G, D), lambda i: (i, 0)),
            pl.BlockSpec((1, NCH, CROWS), lambda i: (i, 0, 0)),
            pl.BlockSpec((1, NCH, CROWS), lambda i: (i, 0, 0)),
        ],
        out_shape=[
            jax.ShapeDtypeStruct((B, D), jnp.float32),
            jax.ShapeDtypeStruct((GRID, NCH, CROWS), jnp.float32),
            jax.ShapeDtypeStruct((GRID, NCH, CROWS), jnp.float32),
        ],
        compiler_params=pltpu.CompilerParams(
            dimension_semantics=("parallel",)),
    )(feats, oh, ph_query, W_phk, b_phk.reshape(1, D), W_phv,
      b_phv.reshape(1, D), ancestry_query, W_ank, b_ank.reshape(1, D),
      W_anv, b_anv.reshape(1, D))
    return (h, wp.reshape(B * N, 1), wa.reshape(B * N, 1))
